# baseline (speedup 1.0000x reference)
import functools

import jax
import jax.numpy as jnp
from jax import lax
from jax.experimental import pallas as pl
from jax.experimental.pallas import tpu as pltpu
from jax.experimental.pallas import tpu_sc as plsc

DIM = 16
LANES = 16
NUM_CORES = 2
NUM_SUBCORES = 16
NUM_WORKERS = NUM_CORES * NUM_SUBCORES
HALF = 128
REG_COEF = 0.001


def _make_sc_fused(batch):
    b_per_w = batch // NUM_WORKERS
    n_halves = b_per_w // HALF
    mesh = plsc.VectorSubcoreMesh(core_axis_name="c", subcore_axis_name="s")

    @functools.partial(
        pl.kernel,
        mesh=mesh,
        out_type=(
            jax.ShapeDtypeStruct((batch,), jnp.float32),
            jax.ShapeDtypeStruct((NUM_WORKERS, LANES), jnp.float32),
        ),
        scratch_types=[
            pltpu.VMEM((b_per_w,), jnp.int32),
            pltpu.VMEM((b_per_w,), jnp.int32),
            pltpu.VMEM((HALF, DIM), jnp.float32),
            pltpu.VMEM((HALF, DIM), jnp.float32),
            pltpu.VMEM((HALF, DIM), jnp.float32),
            pltpu.VMEM((HALF, DIM), jnp.float32),
            pltpu.VMEM((b_per_w,), jnp.float32),
            pltpu.VMEM((LANES,), jnp.float32),
            pltpu.SemaphoreType.DMA,
            pltpu.SemaphoreType.DMA,
            pltpu.SemaphoreType.DMA,
            pltpu.SemaphoreType.DMA,
        ],
        compiler_params=pltpu.CompilerParams(needs_layout_passes=False),
    )
    def sc_kernel(users_hbm, items_hbm, utab_hbm, itab_hbm,
                  inf_out, reg_out,
                  uidx_v, iidx_v, ubuf0, ubuf1, vbuf0, vbuf1,
                  inf_v, racc_v, semu0, semu1, semv0, semv1):
        ubufs, vbufs = (ubuf0, ubuf1), (vbuf0, vbuf1)
        semus, semvs = (semu0, semu1), (semv0, semv1)
        wid = lax.axis_index("s") * NUM_CORES + lax.axis_index("c")
        base = wid * b_per_w

        pltpu.sync_copy(users_hbm.at[pl.ds(base, b_per_w)], uidx_v)
        pltpu.sync_copy(items_hbm.at[pl.ds(base, b_per_w)], iidx_v)

        racc_v[...] = jnp.zeros((LANES,), jnp.float32)
        iota = lax.iota(jnp.int32, LANES)

        def fire(h):
            ubuf, vbuf = ubufs[h % 2], vbufs[h % 2]
            semu, semv = semus[h % 2], semvs[h % 2]

            @pl.loop(0, HALF, step=LANES)
            def _(t):
                uvec = uidx_v[pl.ds(h * HALF + t, LANES)]
                ivec = iidx_v[pl.ds(h * HALF + t, LANES)]
                for k in range(LANES):
                    pltpu.async_copy(utab_hbm.at[uvec[k]], ubuf.at[t + k],
                                     semu)
                    pltpu.async_copy(itab_hbm.at[ivec[k]], vbuf.at[t + k],
                                     semv)

        def drain(h):
            semu, semv = semus[h % 2], semvs[h % 2]

            @pl.loop(0, HALF)
            def _(j):
                pltpu.make_async_copy(utab_hbm.at[0], ubufs[h % 2].at[0],
                                      semu).wait()
                pltpu.make_async_copy(itab_hbm.at[0], vbufs[h % 2].at[0],
                                      semv).wait()

        def compute(h):
            ubuf, vbuf = ubufs[h % 2], vbufs[h % 2]

            @pl.loop(0, HALF, step=LANES)
            def _(t):
                row16 = t + iota
                acc = jnp.zeros((LANES,), jnp.float32)
                rloc = jnp.zeros((LANES,), jnp.float32)
                for l in range(DIM):
                    lane = jnp.full((LANES,), l, jnp.int32)
                    cu16 = plsc.load_gather(ubuf, [row16, lane])
                    cv16 = plsc.load_gather(vbuf, [row16, lane])
                    acc = acc + cu16 * cv16
                    rloc = rloc + (cu16 * cu16 + cv16 * cv16)
                inf_v[pl.ds(h * HALF + t, LANES)] = acc
                racc_v[...] = racc_v[...] + rloc

        fire(0)
        for h in range(n_halves):
            if h + 1 < n_halves:
                fire(h + 1)
            drain(h)
            compute(h)

        pltpu.sync_copy(inf_v, inf_out.at[pl.ds(base, b_per_w)])
        pltpu.sync_copy(racc_v, reg_out.at[wid])

    return sc_kernel


def _reg_body(p_ref, out_ref):
    out_ref[0, 0] = REG_COEF * jnp.sum(p_ref[...])


def kernel(users, items, user_table, item_table):
    batch = users.shape[0]
    users = users.astype(jnp.int32)
    items = items.astype(jnp.int32)
    inf, reg_partials = _make_sc_fused(batch)(
        users, items, user_table, item_table)

    regs = REG_COEF * jnp.sum(reg_partials)
    return inf.reshape(batch, 1), regs
